# Initial kernel scaffold; baseline (speedup 1.0000x reference)
import jax
import jax.numpy as jnp
from jax.experimental import pallas as pl


def kernel(image_embeddings, text_embeddings, vision_indices):
    raise NotImplementedError("write your pallas kernel here")


# speedup vs baseline: 3.5708x; 3.5708x over previous
import functools

import jax
import jax.numpy as jnp
from jax import lax
from jax.experimental import pallas as pl
from jax.experimental.pallas import tpu as pltpu
from jax.experimental.pallas import tpu_sc as plsc

B, S, D = 4, 8192, 2048
N = 1024
NC, NS = 2, 16
NW = NC * NS
ROWS_PER_W = (B * S) // NW
ENT_PER_W = (B * N) // NW
CH = 16
N_COPY = ROWS_PER_W // CH
N_SCAT = ENT_PER_W // CH
BIG = 2**30


def _idx_body(vi_ref, dst_ref, src_ref):
  vi = vi_ref[...]
  rows = vi.shape[0]
  nxt = jnp.concatenate(
      [vi[:, 1:], jnp.full((rows, 1), -1, jnp.int32)], axis=1)
  is_last = vi != nxt
  j = lax.broadcasted_iota(jnp.int32, vi.shape, 1)
  w = jnp.where(is_last, j, BIG)
  k = 1
  while k < vi.shape[1]:
    shifted = jnp.concatenate(
        [w[:, k:], jnp.full((rows, k), BIG, jnp.int32)], axis=1)
    w = jnp.minimum(w, shifted)
    k *= 2
  b = lax.broadcasted_iota(jnp.int32, vi.shape, 0)
  dst_ref[...] = b * S + vi
  src_ref[...] = b * N + w


def _make_indices(vision_indices):
  rows = 8
  vi = jnp.concatenate(
      [vision_indices.astype(jnp.int32),
       jnp.zeros((rows - B, N), jnp.int32)], axis=0)
  dst8, src8 = pl.pallas_call(
      _idx_body,
      out_shape=[
          jax.ShapeDtypeStruct((rows, N), jnp.int32),
          jax.ShapeDtypeStruct((rows, N), jnp.int32),
      ],
  )(vi)
  return dst8[:B].reshape(-1), src8[:B].reshape(-1)


def _sc_body(text_hbm, img_hbm, dst_hbm, src_hbm, out_hbm,
             buf_a, buf_b, dsti_v, srci_v, row0_v,
             sem_la, sem_lb, sem_sa, sem_sb, sem_g, sem_sc):
  c = lax.axis_index("c")
  s = lax.axis_index("s")
  wid = c * NS + s

  bufs = (buf_a, buf_b)
  lsems = (sem_la, sem_lb)
  ssems = (sem_sa, sem_sb)

  base = wid * ROWS_PER_W
  loads = [None] * N_COPY
  stores = [None] * N_COPY
  loads[0] = pltpu.async_copy(
      text_hbm.at[pl.ds(base, CH)], bufs[0], lsems[0])
  for t in range(N_COPY):
    if t + 1 < N_COPY:
      if t >= 1:
        stores[t - 1].wait()
      loads[t + 1] = pltpu.async_copy(
          text_hbm.at[pl.ds(base + (t + 1) * CH, CH)],
          bufs[(t + 1) % 2], lsems[(t + 1) % 2])
    loads[t].wait()
    stores[t] = pltpu.async_copy(
        bufs[t % 2], out_hbm.at[pl.ds(base + t * CH, CH)], ssems[t % 2])
  stores[N_COPY - 2].wait()
  stores[N_COPY - 1].wait()

  plsc.subcore_barrier()

  ebase = wid * ENT_PER_W
  for t in range(N_SCAT):
    off = ebase + t * CH
    pltpu.sync_copy(dst_hbm.at[pl.ds(off, CH)], dsti_v)
    pltpu.sync_copy(src_hbm.at[pl.ds(off, CH)], srci_v)
    pltpu.async_copy(img_hbm.at[srci_v], bufs[t % 2], sem_g).wait()
    pltpu.async_copy(bufs[t % 2], out_hbm.at[dsti_v], sem_sc).wait()

  plsc.subcore_barrier()

  @pl.when(s == 0)
  def _():
    for bb in range(B // NC):
      b = c * (B // NC) + bb
      pltpu.sync_copy(text_hbm.at[pl.ds(b * S, 1)], row0_v)
      pltpu.sync_copy(row0_v, out_hbm.at[pl.ds(b * S, 1)])


@functools.partial(jax.jit, static_argnames=())
def kernel(image_embeddings, text_embeddings, vision_indices):
  text_flat = text_embeddings.reshape(B * S, D)
  img_flat = image_embeddings.reshape(B * N, D)
  dst, src = _make_indices(vision_indices)

  mesh = plsc.VectorSubcoreMesh(
      core_axis_name="c", subcore_axis_name="s",
      num_cores=NC, num_subcores=NS)
  sc = pl.kernel(
      _sc_body,
      out_type=jax.ShapeDtypeStruct((B * S, D), jnp.float32),
      mesh=mesh,
      scratch_types=[
          pltpu.VMEM((CH, D), jnp.float32),
          pltpu.VMEM((CH, D), jnp.float32),
          pltpu.VMEM((CH,), jnp.int32),
          pltpu.VMEM((CH,), jnp.int32),
          pltpu.VMEM((1, D), jnp.float32),
          pltpu.SemaphoreType.DMA,
          pltpu.SemaphoreType.DMA,
          pltpu.SemaphoreType.DMA,
          pltpu.SemaphoreType.DMA,
          pltpu.SemaphoreType.DMA,
          pltpu.SemaphoreType.DMA,
      ],
  )
  out = sc(text_flat, img_flat, dst, src)
  return out.reshape(B, S, D)
